# Initial kernel scaffold; baseline (speedup 1.0000x reference)
import jax
import jax.numpy as jnp
from jax.experimental import pallas as pl


def kernel(x, edge_index, edge_weight, edges, degree, W1, b1, W2, b2, bn_g, bn_b, be_g, be_b, bd_g, bd_b, W0, b0, emb, Wf, bf):
    raise NotImplementedError("write your pallas kernel here")


# speedup vs baseline: 1.0950x; 1.0950x over previous
import functools

import jax
import jax.numpy as jnp
from jax.experimental import pallas as pl
from jax.experimental.pallas import tpu as pltpu

N_NODES = 10000
D = 128
EPS = 1e-5


def _tail_body(h_ref, e_ref, d_ref, Wf_ref, bf_ref, bn_g_ref, bn_b_ref,
               be_g_ref, be_b_ref, bd_g_ref, bd_b_ref, out_ref):
    def bn_relu(v, g, b):
        mu = jnp.sum(v, axis=0, keepdims=True) * (1.0 / N_NODES)
        var = jnp.sum((v - mu) ** 2, axis=0, keepdims=True) * (1.0 / N_NODES)
        return jax.nn.relu(g * (v - mu) * jax.lax.rsqrt(var + EPS) + b)

    h = bn_relu(h_ref[...], bn_g_ref[...], bn_b_ref[...])
    e = bn_relu(e_ref[...], be_g_ref[...], be_b_ref[...])
    d = bn_relu(d_ref[...], bd_g_ref[...], bd_b_ref[...])
    Wf = Wf_ref[...]
    acc = jnp.dot(h, Wf[0:D], preferred_element_type=jnp.float32)
    acc += jnp.dot(e, Wf[D:2 * D], preferred_element_type=jnp.float32)
    acc += jnp.dot(d, Wf[2 * D:3 * D], preferred_element_type=jnp.float32)
    out_ref[...] = acc + bf_ref[...]


def _tail(h, e, d, Wf, bf, bn_g, bn_b, be_g, be_b, bd_g, bd_b):
    return pl.pallas_call(
        _tail_body,
        out_shape=jax.ShapeDtypeStruct((N_NODES, D), jnp.float32),
    )(h, e, d, Wf, bf.reshape(1, D), bn_g.reshape(1, D), bn_b.reshape(1, D),
      be_g.reshape(1, D), be_b.reshape(1, D), bd_g.reshape(1, D),
      bd_b.reshape(1, D))


def kernel(x, edge_index, edge_weight, edges, degree, W1, b1, W2, b2, bn_g,
           bn_b, be_g, be_b, bd_g, bd_b, W0, b0, emb, Wf, bf):
    src, dst = edge_index[0], edge_index[1]
    n = N_NODES
    deg = jax.ops.segment_sum(edge_weight, dst, num_segments=n) + 1.0
    dis = jax.lax.rsqrt(deg)
    coef = dis[src] * edge_weight * dis[dst]
    deg_inv = dis * dis

    def conv(h, W, b):
        hW = h @ W
        msgs = jnp.take(hW, src, axis=0) * coef[:, None]
        out = jax.ops.segment_sum(msgs, dst, num_segments=n)
        return out + deg_inv[:, None] * hW + b

    h = jax.nn.relu(conv(x, W1, b1))
    h = conv(h, W2, b2)
    e = edges @ W0 + b0
    d = jnp.take(emb, degree, axis=0)
    return _tail(h, e, d, Wf, bf, bn_g, bn_b, be_g, be_b, bd_g, bd_b)


# speedup vs baseline: 8.6537x; 7.9028x over previous
import functools

import jax
import jax.numpy as jnp
from jax import lax
from jax.experimental import pallas as pl
from jax.experimental.pallas import tpu as pltpu
from jax.experimental.pallas import tpu_sc as plsc

N = 10000
NP = 10240
D = 128
E_RAW = 320000
E_AUG = E_RAW + N
NT = 32
CH = 128
NCHUNK = -(-E_AUG // (NT * CH))
EW_T = NCHUNK * CH
EA = EW_T * NT
ROWS_T = NP // 16
EPS = 1e-5

_mesh = plsc.VectorSubcoreMesh(core_axis_name="c", subcore_axis_name="s")
_sc_params = pltpu.CompilerParams(needs_layout_passes=False)


def _sc_pre_body(dst_hbm, ew_hbm, ids_hbm, emb_hbm, degp_hbm, draw_hbm,
                 idx_v, ew_v, acc_v, ids_v, erows_v, sem):
    c = lax.axis_index("c")
    s = lax.axis_index("s")
    w = c * 16 + s

    def zbody(i, _):
        acc_v[pl.ds(i * 16, 16)] = jnp.zeros((16,), jnp.float32)
        return 0

    lax.fori_loop(0, NP // 16, zbody, 0)

    base = w * EW_T

    def chunk(g, _):
        e0 = base + g * CH
        pltpu.sync_copy(dst_hbm.at[pl.ds(e0, CH)], idx_v)
        pltpu.sync_copy(ew_hbm.at[pl.ds(e0, CH)], ew_v)
        for j in range(CH // 16):
            dv = idx_v[pl.ds(j * 16, 16)]
            wv = ew_v[pl.ds(j * 16, 16)]
            plsc.addupdate_scatter(acc_v, [dv], wv)
        return 0

    lax.fori_loop(0, NCHUNK, chunk, 0)
    pltpu.sync_copy(acc_v, degp_hbm.at[w])

    nb = w * (NP // NT)

    def echunk(k, _):
        pltpu.sync_copy(ids_hbm.at[pl.ds(nb + k * 80, 80)], ids_v)
        pltpu.async_copy(emb_hbm.at[ids_v], erows_v, sem).wait()
        pltpu.sync_copy(erows_v, draw_hbm.at[pl.ds(nb + k * 80, 80)])
        return 0

    lax.fori_loop(0, 4, echunk, 0)


_sc_pre = pl.kernel(
    _sc_pre_body,
    out_type=(jax.ShapeDtypeStruct((NT, NP), jnp.float32),
              jax.ShapeDtypeStruct((NP, D), jnp.float32)),
    mesh=_mesh,
    compiler_params=_sc_params,
    scratch_types=[
        pltpu.VMEM((CH,), jnp.int32),
        pltpu.VMEM((CH,), jnp.float32),
        pltpu.VMEM((NP,), jnp.float32),
        pltpu.VMEM((80,), jnp.int32),
        pltpu.VMEM((80, D), jnp.float32),
        pltpu.SemaphoreType.DMA,
    ],
)


def _sc_conv_body(xw_hbm, src_hbm, dst_hbm, ew_hbm, dis_hbm, zer_hbm,
                  out_hbm, dis_v, si_v, di_v, ew_v, cf_v, rows_v, acc_sh,
                  sem):
    c = lax.axis_index("c")
    s = lax.axis_index("s")
    w = c * 16 + s

    pltpu.sync_copy(dis_hbm, dis_v)
    pltpu.sync_copy(zer_hbm, acc_sh.at[pl.ds(s * ROWS_T, ROWS_T)])
    plsc.subcore_barrier()

    base = w * EW_T

    def chunk(g, _):
        e0 = base + g * CH
        pltpu.sync_copy(src_hbm.at[pl.ds(e0, CH)], si_v)
        pltpu.sync_copy(dst_hbm.at[pl.ds(e0, CH)], di_v)
        pltpu.sync_copy(ew_hbm.at[pl.ds(e0, CH)], ew_v)
        pltpu.async_copy(xw_hbm.at[si_v], rows_v, sem).wait()
        for j in range(CH // 16):
            sj = si_v[pl.ds(j * 16, 16)]
            dj = di_v[pl.ds(j * 16, 16)]
            wj = ew_v[pl.ds(j * 16, 16)]
            cf = (plsc.load_gather(dis_v, [sj]) * wj
                  * plsc.load_gather(dis_v, [dj]))
            cf_v[pl.ds(j * 16, 16)] = cf

        def scale(e, _):
            fv = plsc.load_gather(cf_v, [jnp.broadcast_to(e, (16,))])
            for jj in range(D // 16):
                rows_v[e, pl.ds(jj * 16, 16)] = (
                    rows_v[e, pl.ds(jj * 16, 16)] * fv)
            return 0

        lax.fori_loop(0, CH, scale, 0)
        pltpu.sync_copy(rows_v, acc_sh.at[di_v], add=True)
        return 0

    lax.fori_loop(0, NCHUNK, chunk, 0)
    plsc.subcore_barrier()
    pltpu.sync_copy(acc_sh.at[pl.ds(s * ROWS_T, ROWS_T)],
                    out_hbm.at[c, pl.ds(s * ROWS_T, ROWS_T)])


_sc_conv = pl.kernel(
    _sc_conv_body,
    out_type=jax.ShapeDtypeStruct((2, NP, D), jnp.float32),
    mesh=_mesh,
    compiler_params=_sc_params,
    scratch_types=[
        pltpu.VMEM((NP,), jnp.float32),
        pltpu.VMEM((CH,), jnp.int32),
        pltpu.VMEM((CH,), jnp.int32),
        pltpu.VMEM((CH,), jnp.float32),
        pltpu.VMEM((CH,), jnp.float32),
        pltpu.VMEM((CH, D), jnp.float32),
        pltpu.VMEM_SHARED((NP, D), jnp.float32),
        pltpu.SemaphoreType.DMA,
    ],
)


def _tc1_body(degp_ref, x_ref, W1_ref, dis_ref, xw1_ref):
    deg = jnp.sum(degp_ref[...], axis=0, keepdims=True)
    dis_ref[...] = jnp.where(deg > 0, jax.lax.rsqrt(deg), 0.0)
    xw1_ref[...] = jnp.dot(x_ref[...], W1_ref[...],
                           preferred_element_type=jnp.float32)


def _tc2_body(part_ref, b1_ref, W2_ref, xw2_ref):
    h = part_ref[0, 0:N, :] + part_ref[1, 0:N, :] + b1_ref[...]
    h = jax.nn.relu(h)
    xw2_ref[...] = jnp.dot(h, W2_ref[...],
                           preferred_element_type=jnp.float32)


def _bn_relu(v, g, b):
    mu = jnp.sum(v, axis=0, keepdims=True) * (1.0 / N)
    var = jnp.sum((v - mu) ** 2, axis=0, keepdims=True) * (1.0 / N)
    return jax.nn.relu(g * (v - mu) * jax.lax.rsqrt(var + EPS) + b)


def _tc3_body(part_ref, b2_ref, draw_ref, edges_ref, W0_ref, b0_ref,
              bn_g_ref, bn_b_ref, be_g_ref, be_b_ref, bd_g_ref, bd_b_ref,
              Wf_ref, bf_ref, out_ref):
    h = part_ref[0, 0:N, :] + part_ref[1, 0:N, :] + b2_ref[...]
    h = _bn_relu(h, bn_g_ref[...], bn_b_ref[...])
    e = edges_ref[...] * W0_ref[...] + b0_ref[...]
    e = _bn_relu(e, be_g_ref[...], be_b_ref[...])
    d = _bn_relu(draw_ref[0:N, :], bd_g_ref[...], bd_b_ref[...])
    Wf = Wf_ref[...]
    acc = jnp.dot(h, Wf[0:D], preferred_element_type=jnp.float32)
    acc += jnp.dot(e, Wf[D:2 * D], preferred_element_type=jnp.float32)
    acc += jnp.dot(d, Wf[2 * D:3 * D], preferred_element_type=jnp.float32)
    out_ref[...] = acc + bf_ref[...]


def kernel(x, edge_index, edge_weight, edges, degree, W1, b1, W2, b2, bn_g,
           bn_b, be_g, be_b, bd_g, bd_b, W0, b0, emb, Wf, bf):
    src, dst = edge_index[0], edge_index[1]
    idt = src.dtype
    pad = EA - E_AUG
    loop = jnp.arange(N, dtype=idt)
    srcA = jnp.concatenate([src, loop, jnp.zeros((pad,), idt)])
    dstA = jnp.concatenate([dst, loop, jnp.zeros((pad,), idt)])
    ewA = jnp.concatenate([edge_weight, jnp.ones((N,), jnp.float32),
                           jnp.zeros((pad,), jnp.float32)])
    ids_p = jnp.concatenate([degree, jnp.zeros((NP - N,), degree.dtype)])
    zer = jnp.zeros((ROWS_T, D), jnp.float32)

    degp, d_raw = _sc_pre(dstA, ewA, ids_p, emb)

    dis2d, xw1 = pl.pallas_call(
        _tc1_body,
        out_shape=(jax.ShapeDtypeStruct((1, NP), jnp.float32),
                   jax.ShapeDtypeStruct((N, D), jnp.float32)),
    )(degp, x, W1)
    dis = dis2d.reshape(NP)

    part1 = _sc_conv(xw1, srcA, dstA, ewA, dis, zer)

    xw2 = pl.pallas_call(
        _tc2_body,
        out_shape=jax.ShapeDtypeStruct((N, D), jnp.float32),
    )(part1, b1.reshape(1, D), W2)

    part2 = _sc_conv(xw2, srcA, dstA, ewA, dis, zer)

    return pl.pallas_call(
        _tc3_body,
        out_shape=jax.ShapeDtypeStruct((N, D), jnp.float32),
    )(part2, b2.reshape(1, D), d_raw, edges, W0, b0.reshape(1, D),
      bn_g.reshape(1, D), bn_b.reshape(1, D), be_g.reshape(1, D),
      be_b.reshape(1, D), bd_g.reshape(1, D), bd_b.reshape(1, D),
      Wf, bf.reshape(1, D))
